# Initial kernel scaffold; baseline (speedup 1.0000x reference)
import jax
import jax.numpy as jnp
from jax.experimental import pallas as pl


def kernel(gps_idx, table):
    raise NotImplementedError("write your pallas kernel here")


# speedup vs baseline: 1.7970x; 1.7970x over previous
import functools

import jax
import jax.numpy as jnp
from jax import lax
from jax.experimental import pallas as pl
from jax.experimental.pallas import tpu as pltpu
from jax.experimental.pallas import tpu_sc as plsc

NC = 2
NS = 16
NW = NC * NS

BATCH = 16384
HIST = 50
EMBED_DIM = 64
B = BATCH * HIST
B_PER_W = B // NW
CHUNK = 512
N_CHUNKS = B_PER_W // CHUNK


def _make_sc_gather():
    mesh = plsc.VectorSubcoreMesh(core_axis_name="c", subcore_axis_name="s")

    @functools.partial(
        pl.kernel,
        out_type=jax.ShapeDtypeStruct((B, EMBED_DIM), jnp.float32),
        mesh=mesh,
        scratch_types=[
            pltpu.VMEM((CHUNK,), jnp.int32),
            pltpu.VMEM((CHUNK, EMBED_DIM), jnp.float32),
            pltpu.SemaphoreType.DMA,
        ],
        compiler_params=pltpu.CompilerParams(use_tc_tiling_on_sc=False),
    )
    def k(idx_hbm, table_hbm, out_hbm, idx_v, rows_v, sem):
        wid = lax.axis_index("s") * NC + lax.axis_index("c")
        base = wid * B_PER_W

        @pl.loop(0, N_CHUNKS)
        def _(g):
            start = base + g * CHUNK
            pltpu.sync_copy(idx_hbm.at[pl.ds(start, CHUNK)], idx_v)
            pltpu.async_copy(table_hbm.at[idx_v], rows_v, sem).wait()
            pltpu.sync_copy(rows_v, out_hbm.at[pl.ds(start, CHUNK)])

    return k


_sc_gather = _make_sc_gather()


@jax.jit
def kernel(gps_idx, table):
    idx_flat = gps_idx.astype(jnp.int32).reshape(-1)
    out = _sc_gather(idx_flat, table)
    return out.reshape(BATCH, HIST, EMBED_DIM)


# speedup vs baseline: 1.8727x; 1.0421x over previous
import functools

import jax
import jax.numpy as jnp
from jax import lax
from jax.experimental import pallas as pl
from jax.experimental.pallas import tpu as pltpu
from jax.experimental.pallas import tpu_sc as plsc

NC = 2
NS = 16
NW = NC * NS

BATCH = 16384
HIST = 50
EMBED_DIM = 64
B = BATCH * HIST
B_PER_W = B // NW
CHUNK = 800
N_CHUNKS = B_PER_W // CHUNK


def _make_sc_gather():
    mesh = plsc.VectorSubcoreMesh(core_axis_name="c", subcore_axis_name="s")

    @functools.partial(
        pl.kernel,
        out_type=jax.ShapeDtypeStruct((B, EMBED_DIM), jnp.float32),
        mesh=mesh,
        scratch_types=[
            pltpu.VMEM((B_PER_W,), jnp.int32),
            pltpu.VMEM((2 * CHUNK, EMBED_DIM), jnp.float32),
            pltpu.SemaphoreType.DMA,
            pltpu.SemaphoreType.DMA,
        ],
        compiler_params=pltpu.CompilerParams(use_tc_tiling_on_sc=False),
    )
    def k(idx_hbm, table_hbm, out_hbm, idx_v, rows_v, gsem, wsem):
        wid = lax.axis_index("s") * NC + lax.axis_index("c")
        base = wid * B_PER_W

        pltpu.sync_copy(idx_hbm.at[pl.ds(base, B_PER_W)], idx_v)

        def rows_buf(g):
            return rows_v.at[pl.ds((g % 2) * CHUNK, CHUNK)]

        def start_gather(g):
            pltpu.async_copy(
                table_hbm.at[idx_v.at[pl.ds(g * CHUNK, CHUNK)]],
                rows_buf(g), gsem)

        def wait_gather(g):
            pltpu.make_async_copy(
                table_hbm.at[idx_v.at[pl.ds(g * CHUNK, CHUNK)]],
                rows_buf(g), gsem).wait()

        def start_wb(g):
            pltpu.async_copy(
                rows_buf(g), out_hbm.at[pl.ds(base + g * CHUNK, CHUNK)], wsem)

        def wait_wb(g):
            pltpu.make_async_copy(
                rows_buf(g), out_hbm.at[pl.ds(base + g * CHUNK, CHUNK)],
                wsem).wait()

        start_gather(0)
        wait_gather(0)
        start_gather(1)
        start_wb(0)

        @pl.loop(1, N_CHUNKS - 1)
        def _(g):
            wait_gather(g)
            wait_wb(g - 1)
            start_gather(g + 1)
            start_wb(g)

        wait_gather(N_CHUNKS - 1)
        wait_wb(N_CHUNKS - 2)
        start_wb(N_CHUNKS - 1)
        wait_wb(N_CHUNKS - 1)

    return k


_sc_gather = _make_sc_gather()


@jax.jit
def kernel(gps_idx, table):
    idx_flat = gps_idx.astype(jnp.int32).reshape(-1)
    out = _sc_gather(idx_flat, table)
    return out.reshape(BATCH, HIST, EMBED_DIM)


# speedup vs baseline: 2.5283x; 1.3501x over previous
import functools

import jax
import jax.numpy as jnp
from jax import lax
from jax.experimental import pallas as pl
from jax.experimental.pallas import tpu as pltpu
from jax.experimental.pallas import tpu_sc as plsc

NC = 2
NS = 16
NW = NC * NS

BATCH = 16384
HIST = 50
EMBED_DIM = 64
HP = 56
DP = 128
B_PER_W = BATCH // NW
NB = 16
CHUNK = NB * HIST
N_CHUNKS = B_PER_W // NB


def _make_sc_gather():
    mesh = plsc.VectorSubcoreMesh(core_axis_name="c", subcore_axis_name="s")

    @functools.partial(
        pl.kernel,
        out_type=jax.ShapeDtypeStruct((BATCH, HP, DP), jnp.float32),
        mesh=mesh,
        scratch_types=[
            pltpu.VMEM((B_PER_W * HIST,), jnp.int32),
            pltpu.VMEM((2 * CHUNK, EMBED_DIM), jnp.float32),
            pltpu.SemaphoreType.DMA,
            pltpu.SemaphoreType.DMA,
        ],
        compiler_params=pltpu.CompilerParams(use_tc_tiling_on_sc=False),
    )
    def k(idx_hbm, table_hbm, out_hbm, idx_v, rows_v, gsem, wsem):
        wid = lax.axis_index("s") * NC + lax.axis_index("c")
        wb0 = wid * B_PER_W

        pltpu.sync_copy(idx_hbm.at[pl.ds(wb0 * HIST, B_PER_W * HIST)], idx_v)

        def rows_buf(g):
            return rows_v.at[pl.ds((g % 2) * CHUNK, CHUNK)]

        def start_gather(g):
            pltpu.async_copy(
                table_hbm.at[idx_v.at[pl.ds(g * CHUNK, CHUNK)]],
                rows_buf(g), gsem)

        def wait_gather(g):
            pltpu.make_async_copy(
                table_hbm.at[idx_v.at[pl.ds(g * CHUNK, CHUNK)]],
                rows_buf(g), gsem).wait()

        def start_wb(g):
            b = (g % 2) * CHUNK
            for j in range(NB):
                pltpu.async_copy(
                    rows_v.at[pl.ds(b + j * HIST, HIST)],
                    out_hbm.at[wb0 + g * NB + j, pl.ds(0, HIST),
                               pl.ds(0, EMBED_DIM)],
                    wsem)

        def wait_wb(g):
            b = (g % 2) * CHUNK
            for j in range(NB):
                pltpu.make_async_copy(
                    rows_v.at[pl.ds(b + j * HIST, HIST)],
                    out_hbm.at[wb0 + g * NB + j, pl.ds(0, HIST),
                               pl.ds(0, EMBED_DIM)],
                    wsem).wait()

        start_gather(0)
        wait_gather(0)
        start_gather(1)
        start_wb(0)

        @pl.loop(1, N_CHUNKS - 1)
        def _(g):
            wait_gather(g)
            wait_wb(g - 1)
            start_gather(g + 1)
            start_wb(g)

        wait_gather(N_CHUNKS - 1)
        wait_wb(N_CHUNKS - 2)
        start_wb(N_CHUNKS - 1)
        wait_wb(N_CHUNKS - 1)

    return k


_sc_gather = _make_sc_gather()


@jax.jit
def kernel(gps_idx, table):
    idx_flat = gps_idx.astype(jnp.int32).reshape(-1)
    out3 = _sc_gather(idx_flat, table)
    return out3[:, :HIST, :EMBED_DIM]


# speedup vs baseline: 2.5303x; 1.0008x over previous
import functools

import jax
import jax.numpy as jnp
from jax import lax
from jax.experimental import pallas as pl
from jax.experimental.pallas import tpu as pltpu
from jax.experimental.pallas import tpu_sc as plsc

NC = 2
NS = 16
NW = NC * NS

BATCH = 16384
HIST = 50
EMBED_DIM = 64
HP = 56
DP = 128
B_PER_W = BATCH // NW
NB = 8
CHUNK = NB * HIST
N_CHUNKS = B_PER_W // NB


def _make_sc_gather():
    mesh = plsc.VectorSubcoreMesh(core_axis_name="c", subcore_axis_name="s")

    @functools.partial(
        pl.kernel,
        out_type=jax.ShapeDtypeStruct((BATCH, HP, DP), jnp.float32),
        mesh=mesh,
        scratch_types=[
            pltpu.VMEM((B_PER_W * HIST,), jnp.int32),
            pltpu.VMEM((3 * CHUNK, EMBED_DIM), jnp.float32),
            pltpu.SemaphoreType.DMA,
            pltpu.SemaphoreType.DMA,
        ],
        compiler_params=pltpu.CompilerParams(use_tc_tiling_on_sc=False),
    )
    def k(idx_hbm, table_hbm, out_hbm, idx_v, rows_v, gsem, wsem):
        wid = lax.axis_index("s") * NC + lax.axis_index("c")
        wb0 = wid * B_PER_W

        pltpu.sync_copy(idx_hbm.at[pl.ds(wb0 * HIST, B_PER_W * HIST)], idx_v)

        def rows_buf(g):
            return rows_v.at[pl.ds((g % 3) * CHUNK, CHUNK)]

        def start_gather(g):
            pltpu.async_copy(
                table_hbm.at[idx_v.at[pl.ds(g * CHUNK, CHUNK)]],
                rows_buf(g), gsem)

        def wait_gather(g):
            pltpu.make_async_copy(
                table_hbm.at[idx_v.at[pl.ds(g * CHUNK, CHUNK)]],
                rows_buf(g), gsem).wait()

        def start_wb(g):
            b = (g % 3) * CHUNK
            for j in range(NB):
                pltpu.async_copy(
                    rows_v.at[pl.ds(b + j * HIST, HIST)],
                    out_hbm.at[wb0 + g * NB + j, pl.ds(0, HIST),
                               pl.ds(0, EMBED_DIM)],
                    wsem)

        def wait_wb(g):
            b = (g % 3) * CHUNK
            for j in range(NB):
                pltpu.make_async_copy(
                    rows_v.at[pl.ds(b + j * HIST, HIST)],
                    out_hbm.at[wb0 + g * NB + j, pl.ds(0, HIST),
                               pl.ds(0, EMBED_DIM)],
                    wsem).wait()

        start_gather(0)
        start_gather(1)
        wait_gather(0)
        start_gather(2)
        start_wb(0)

        @pl.loop(1, N_CHUNKS - 2)
        def _(g):
            wait_gather(g)
            wait_wb(g - 1)
            start_gather(g + 2)
            start_wb(g)

        wait_gather(N_CHUNKS - 2)
        wait_wb(N_CHUNKS - 3)
        start_wb(N_CHUNKS - 2)
        wait_gather(N_CHUNKS - 1)
        wait_wb(N_CHUNKS - 2)
        start_wb(N_CHUNKS - 1)
        wait_wb(N_CHUNKS - 1)

    return k


_sc_gather = _make_sc_gather()


@jax.jit
def kernel(gps_idx, table):
    idx_flat = gps_idx.astype(jnp.int32).reshape(-1)
    out3 = _sc_gather(idx_flat, table)
    return out3[:, :HIST, :EMBED_DIM]
